# baseline (speedup 1.0000x reference)
import functools

import jax
import jax.numpy as jnp
from jax import lax
from jax.experimental import pallas as pl
from jax.experimental.pallas import tpu as pltpu
from jax.experimental.pallas import tpu_sc as plsc

N = 10000
E = 320000
D = 128
NC = 2
NS = 16
NTILES = NC * NS
NP = ((N + 255) // 256) * 256
RPT = NP // NS
C = 125
NB = 10
BC = 8
CH = NB * BC
EPT = CH * C
BLK = 1024
GRID = (N + BLK - 1) // BLK

_mesh = plsc.VectorSubcoreMesh(core_axis_name="c", subcore_axis_name="s")


@functools.partial(
    pl.kernel,
    mesh=_mesh,
    out_type=jax.ShapeDtypeStruct((NC, NP), jnp.float32),
    scratch_types=[
        pltpu.VMEM((NB, BC, C), jnp.int32),
        pltpu.VMEM((128,), jnp.float32),
        pltpu.VMEM((RPT,), jnp.float32),
        pltpu.VMEM_SHARED((NP,), jnp.float32),
    ],
)
def _deg_sc(eS_hbm, out_hbm, dst_v, ones_v, zbuf, acc):
    cid = lax.axis_index("c")
    sid = lax.axis_index("s")
    wid = sid * NC + cid

    def fill_ones(i, carry):
        ones_v[pl.ds(i * 16, 16)] = jnp.ones((16,), jnp.float32)
        return carry

    lax.fori_loop(0, 8, fill_ones, 0)

    def fill_zeros(i, carry):
        zbuf[pl.ds(i * 16, 16)] = jnp.zeros((16,), jnp.float32)
        return carry

    lax.fori_loop(0, RPT // 16, fill_zeros, 0)
    pltpu.sync_copy(zbuf, acc.at[pl.ds(sid * RPT, RPT)])
    pltpu.sync_copy(eS_hbm.at[1, wid], dst_v)
    plsc.subcore_barrier()

    ones = ones_v.at[pl.ds(0, C)]

    def body(j, carry):
        for r in range(BC):
            pltpu.sync_copy(ones, acc.at[dst_v.at[j, r]], add=True)
        return carry

    lax.fori_loop(0, NB, body, 0)
    plsc.subcore_barrier()
    pltpu.sync_copy(acc.at[pl.ds(sid * RPT, RPT)],
                    out_hbm.at[cid, pl.ds(sid * RPT, RPT)])


def _mm_body(deg_ref, x_ref, w_ref, y_ref):
    d = deg_ref[0] + deg_ref[1] + 1.0
    dinv = lax.rsqrt(jnp.maximum(d, 1e-12))
    xw = jnp.dot(x_ref[...], w_ref[...], preferred_element_type=jnp.float32)
    y_ref[...] = xw * dinv[:, None]


def _mm_tc(degp, xp, W):
    return pl.pallas_call(
        _mm_body,
        grid=(GRID,),
        in_specs=[
            pl.BlockSpec((NC, BLK), lambda i: (0, i)),
            pl.BlockSpec((BLK, D), lambda i: (i, 0)),
            pl.BlockSpec((D, D), lambda i: (0, 0)),
        ],
        out_specs=pl.BlockSpec((BLK, D), lambda i: (i, 0)),
        out_shape=jax.ShapeDtypeStruct((N, D), jnp.float32),
    )(degp, xp, W)


@functools.partial(
    pl.kernel,
    mesh=_mesh,
    out_type=jax.ShapeDtypeStruct((NC, NP, D), jnp.float32),
    scratch_types=[
        pltpu.VMEM((NB, BC, C), jnp.int32),
        pltpu.VMEM((BC, C), jnp.int32),
        pltpu.VMEM((BC, C), jnp.int32),
        pltpu.VMEM((C, D), jnp.float32),
        pltpu.VMEM((C, D), jnp.float32),
        pltpu.VMEM((16, D), jnp.float32),
        pltpu.VMEM_SHARED((NP, D), jnp.float32),
        pltpu.SemaphoreType.DMA,
        pltpu.SemaphoreType.DMA,
        pltpu.SemaphoreType.DMA,
        pltpu.SemaphoreType.DMA,
    ],
)
def _scat_sc(y_hbm, eS_hbm, out_hbm, src_v, dbuf_a, dbuf_b,
             rows_a, rows_b, zbuf, acc, sem_a, sem_b, dsem_a, dsem_b):
    cid = lax.axis_index("c")
    sid = lax.axis_index("s")
    wid = sid * NC + cid

    for i in range(16):
        for j in range(D // 16):
            zbuf[i, pl.ds(j * 16, 16)] = jnp.zeros((16,), jnp.float32)

    def zero_rows(i, carry):
        pltpu.sync_copy(zbuf, acc.at[pl.ds(sid * RPT + i * 16, 16)])
        return carry

    lax.fori_loop(0, RPT // 16, zero_rows, 0)
    pltpu.sync_copy(eS_hbm.at[0, wid], src_v)
    plsc.subcore_barrier()

    rows = (rows_a, rows_b)
    sems = (sem_a, sem_b)

    H = 64

    def gstart(a, r, par):
        buf = rows[par]
        pltpu.async_copy(y_hbm.at[src_v.at[a, r, pl.ds(0, H)]],
                         buf.at[pl.ds(0, H)], sems[par])
        pltpu.async_copy(y_hbm.at[src_v.at[a, r, pl.ds(H, C - H)]],
                         buf.at[pl.ds(H, C - H)], sems[par])

    def gwait(a, r, par):
        buf = rows[par]
        pltpu.make_async_copy(y_hbm.at[src_v.at[a, r, pl.ds(0, H)]],
                              buf.at[pl.ds(0, H)], sems[par]).wait()
        pltpu.make_async_copy(y_hbm.at[src_v.at[a, r, pl.ds(H, C - H)]],
                              buf.at[pl.ds(H, C - H)], sems[par]).wait()

    def dstart(a, dbuf, dsem):
        pltpu.async_copy(eS_hbm.at[1, wid, a], dbuf, dsem)

    def dwait(a, dbuf, dsem):
        pltpu.make_async_copy(eS_hbm.at[1, wid, a], dbuf, dsem).wait()

    def run_block(a, dbuf):
        for k in range(BC):
            if k < BC - 1:
                gstart(a, k + 1, (k + 1) % 2)
            else:
                gstart(jnp.minimum(a + 1, NB - 1), 0, 0)
            gwait(a, k, k % 2)
            pltpu.sync_copy(rows[k % 2], acc.at[dbuf.at[k]], add=True)

    dstart(0, dbuf_a, dsem_a)
    dwait(0, dbuf_a, dsem_a)
    gstart(0, 0, 0)

    def body(j, carry):
        a0 = 2 * j
        dstart(a0 + 1, dbuf_b, dsem_b)
        run_block(a0, dbuf_a)
        dwait(a0 + 1, dbuf_b, dsem_b)
        dstart(jnp.minimum(a0 + 2, NB - 1), dbuf_a, dsem_a)
        run_block(a0 + 1, dbuf_b)
        dwait(jnp.minimum(a0 + 2, NB - 1), dbuf_a, dsem_a)
        return carry

    lax.fori_loop(0, NB // 2, body, 0)
    gwait(NB - 1, 0, 0)
    plsc.subcore_barrier()
    pltpu.sync_copy(acc.at[pl.ds(sid * RPT, RPT)],
                    out_hbm.at[cid, pl.ds(sid * RPT, RPT)])


def _fin_body(sp_ref, y_ref, deg_ref, b_ref, o_ref):
    d = deg_ref[0] + deg_ref[1] + 1.0
    dinv = lax.rsqrt(jnp.maximum(d, 1e-12))
    s = sp_ref[0] + sp_ref[1] + y_ref[...]
    o_ref[...] = dinv[:, None] * s + b_ref[0][None, :]


def _fin_tc(Sp, y, degp, b2):
    return pl.pallas_call(
        _fin_body,
        grid=(GRID,),
        in_specs=[
            pl.BlockSpec((NC, BLK, D), lambda i: (0, i, 0)),
            pl.BlockSpec((BLK, D), lambda i: (i, 0)),
            pl.BlockSpec((NC, BLK), lambda i: (0, i)),
            pl.BlockSpec((1, D), lambda i: (0, 0)),
        ],
        out_specs=pl.BlockSpec((BLK, D), lambda i: (i, 0)),
        out_shape=jax.ShapeDtypeStruct((N, D), jnp.float32),
    )(Sp, y, degp, b2)


def kernel(x, edge_index, W, b):
    eS = edge_index.reshape(2, NTILES, NB, BC, C)
    degp = _deg_sc(eS)
    y = _mm_tc(degp, x, W)
    Sp = _scat_sc(y, eS)
    return _fin_tc(Sp, y, degp, b.reshape(1, D))


# speedup vs baseline: 1.0192x; 1.0192x over previous
import functools

import jax
import jax.numpy as jnp
from jax import lax
from jax.experimental import pallas as pl
from jax.experimental.pallas import tpu as pltpu
from jax.experimental.pallas import tpu_sc as plsc

N = 10000
E = 320000
D = 128
NC = 2
NS = 16
NTILES = NC * NS
NP = ((N + 255) // 256) * 256
RPT = NP // NS
C = 125
NB = 10
BC = 8
CH = NB * BC
EPT = CH * C
BLK = 1024
GRID = (N + BLK - 1) // BLK

_mesh = plsc.VectorSubcoreMesh(core_axis_name="c", subcore_axis_name="s")


@functools.partial(
    pl.kernel,
    mesh=_mesh,
    out_type=jax.ShapeDtypeStruct((NC, 2, NP), jnp.float32),
    scratch_types=[
        pltpu.VMEM((NB, BC, C), jnp.int32),
        pltpu.VMEM((128,), jnp.float32),
        pltpu.VMEM((RPT,), jnp.float32),
        pltpu.VMEM_SHARED((NP,), jnp.float32),
        pltpu.VMEM_SHARED((NP,), jnp.float32),
        pltpu.SemaphoreType.DMA,
        pltpu.SemaphoreType.DMA,
    ],
)
def _deg_sc(eS_hbm, out_hbm, dst_v, ones_v, zbuf, acc0, acc1, sem_a, sem_b):
    cid = lax.axis_index("c")
    sid = lax.axis_index("s")
    wid = sid * NC + cid

    def fill_ones(i, carry):
        ones_v[pl.ds(i * 16, 16)] = jnp.ones((16,), jnp.float32)
        return carry

    lax.fori_loop(0, 8, fill_ones, 0)

    def fill_zeros(i, carry):
        zbuf[pl.ds(i * 16, 16)] = jnp.zeros((16,), jnp.float32)
        return carry

    lax.fori_loop(0, RPT // 16, fill_zeros, 0)
    pltpu.sync_copy(zbuf, acc0.at[pl.ds(sid * RPT, RPT)])
    pltpu.sync_copy(zbuf, acc1.at[pl.ds(sid * RPT, RPT)])
    pltpu.sync_copy(eS_hbm.at[1, wid], dst_v)
    plsc.subcore_barrier()

    ones = ones_v.at[pl.ds(0, C)]
    accs = (acc0, acc1)
    sems = (sem_a, sem_b)

    def sstart(a, r, par):
        pltpu.async_copy(ones, accs[par].at[dst_v.at[a, r]], sems[par])

    def swait(a, r, par):
        pltpu.make_async_copy(ones, accs[par].at[dst_v.at[a, r]],
                              sems[par]).wait()

    sstart(0, 0, 0)

    def body(i, carry):
        c0 = 2 * i
        a0, r0 = c0 // BC, c0 % BC
        a1, r1 = (c0 + 1) // BC, (c0 + 1) % BC
        a2, r2 = (c0 + 2) // BC, (c0 + 2) % BC
        sstart(a1, r1, 1)
        swait(a0, r0, 0)
        sstart(a2, r2, 0)
        swait(a1, r1, 1)
        return carry

    lax.fori_loop(0, (NB * BC - 2) // 2, body, 0)
    sstart(NB - 1, BC - 1, 1)
    swait(NB - 1, BC - 2, 0)
    swait(NB - 1, BC - 1, 1)
    plsc.subcore_barrier()
    pltpu.sync_copy(acc0.at[pl.ds(sid * RPT, RPT)],
                    out_hbm.at[cid, 0, pl.ds(sid * RPT, RPT)])
    pltpu.sync_copy(acc1.at[pl.ds(sid * RPT, RPT)],
                    out_hbm.at[cid, 1, pl.ds(sid * RPT, RPT)])


def _mm_body(deg_ref, x_ref, w_ref, y_ref):
    d = deg_ref[0, 0] + deg_ref[0, 1] + deg_ref[1, 0] + deg_ref[1, 1] + 1.0
    dinv = lax.rsqrt(jnp.maximum(d, 1e-12))
    xw = jnp.dot(x_ref[...], w_ref[...], preferred_element_type=jnp.float32)
    y_ref[...] = xw * dinv[:, None]


def _mm_tc(degp, xp, W):
    return pl.pallas_call(
        _mm_body,
        grid=(GRID,),
        in_specs=[
            pl.BlockSpec((NC, 2, BLK), lambda i: (0, 0, i)),
            pl.BlockSpec((BLK, D), lambda i: (i, 0)),
            pl.BlockSpec((D, D), lambda i: (0, 0)),
        ],
        out_specs=pl.BlockSpec((BLK, D), lambda i: (i, 0)),
        out_shape=jax.ShapeDtypeStruct((N, D), jnp.float32),
    )(degp, xp, W)


@functools.partial(
    pl.kernel,
    mesh=_mesh,
    out_type=jax.ShapeDtypeStruct((NC, NP, D), jnp.float32),
    scratch_types=[
        pltpu.VMEM((NB, BC, C), jnp.int32),
        pltpu.VMEM((BC, C), jnp.int32),
        pltpu.VMEM((BC, C), jnp.int32),
        pltpu.VMEM((C, D), jnp.float32),
        pltpu.VMEM((C, D), jnp.float32),
        pltpu.VMEM((16, D), jnp.float32),
        pltpu.VMEM_SHARED((NP, D), jnp.float32),
        pltpu.SemaphoreType.DMA,
        pltpu.SemaphoreType.DMA,
        pltpu.SemaphoreType.DMA,
        pltpu.SemaphoreType.DMA,
    ],
)
def _scat_sc(y_hbm, eS_hbm, out_hbm, src_v, dbuf_a, dbuf_b,
             rows_a, rows_b, zbuf, acc, sem_a, sem_b, dsem_a, dsem_b):
    cid = lax.axis_index("c")
    sid = lax.axis_index("s")
    wid = sid * NC + cid

    for i in range(16):
        for j in range(D // 16):
            zbuf[i, pl.ds(j * 16, 16)] = jnp.zeros((16,), jnp.float32)

    def zero_rows(i, carry):
        pltpu.sync_copy(zbuf, acc.at[pl.ds(sid * RPT + i * 16, 16)])
        return carry

    lax.fori_loop(0, RPT // 16, zero_rows, 0)
    pltpu.sync_copy(eS_hbm.at[0, wid], src_v)
    plsc.subcore_barrier()

    rows = (rows_a, rows_b)
    sems = (sem_a, sem_b)

    def gstart(a, r, par):
        pltpu.async_copy(y_hbm.at[src_v.at[a, r]], rows[par], sems[par])

    def gwait(a, r, par):
        pltpu.make_async_copy(y_hbm.at[src_v.at[a, r]], rows[par],
                              sems[par]).wait()

    def dstart(a, dbuf, dsem):
        pltpu.async_copy(eS_hbm.at[1, wid, a], dbuf, dsem)

    def dwait(a, dbuf, dsem):
        pltpu.make_async_copy(eS_hbm.at[1, wid, a], dbuf, dsem).wait()

    def run_block(a, dbuf):
        for k in range(BC):
            if k < BC - 1:
                gstart(a, k + 1, (k + 1) % 2)
            else:
                gstart(jnp.minimum(a + 1, NB - 1), 0, 0)
            gwait(a, k, k % 2)
            pltpu.sync_copy(rows[k % 2], acc.at[dbuf.at[k]], add=True)

    dstart(0, dbuf_a, dsem_a)
    dwait(0, dbuf_a, dsem_a)
    gstart(0, 0, 0)

    def body(j, carry):
        a0 = 2 * j
        dstart(a0 + 1, dbuf_b, dsem_b)
        run_block(a0, dbuf_a)
        dwait(a0 + 1, dbuf_b, dsem_b)
        dstart(jnp.minimum(a0 + 2, NB - 1), dbuf_a, dsem_a)
        run_block(a0 + 1, dbuf_b)
        dwait(jnp.minimum(a0 + 2, NB - 1), dbuf_a, dsem_a)
        return carry

    lax.fori_loop(0, NB // 2, body, 0)
    gwait(NB - 1, 0, 0)
    plsc.subcore_barrier()
    pltpu.sync_copy(acc.at[pl.ds(sid * RPT, RPT)],
                    out_hbm.at[cid, pl.ds(sid * RPT, RPT)])


def _fin_body(sp_ref, y_ref, deg_ref, b_ref, o_ref):
    d = deg_ref[0, 0] + deg_ref[0, 1] + deg_ref[1, 0] + deg_ref[1, 1] + 1.0
    dinv = lax.rsqrt(jnp.maximum(d, 1e-12))
    s = sp_ref[0] + sp_ref[1] + y_ref[...]
    o_ref[...] = dinv[:, None] * s + b_ref[0][None, :]


def _fin_tc(Sp, y, degp, b2):
    return pl.pallas_call(
        _fin_body,
        grid=(GRID,),
        in_specs=[
            pl.BlockSpec((NC, BLK, D), lambda i: (0, i, 0)),
            pl.BlockSpec((BLK, D), lambda i: (i, 0)),
            pl.BlockSpec((NC, 2, BLK), lambda i: (0, 0, i)),
            pl.BlockSpec((1, D), lambda i: (0, 0)),
        ],
        out_specs=pl.BlockSpec((BLK, D), lambda i: (i, 0)),
        out_shape=jax.ShapeDtypeStruct((N, D), jnp.float32),
    )(Sp, y, degp, b2)


def kernel(x, edge_index, W, b):
    eS = edge_index.reshape(2, NTILES, NB, BC, C)
    degp = _deg_sc(eS)
    y = _mm_tc(degp, x, W)
    Sp = _scat_sc(y, eS)
    return _fin_tc(Sp, y, degp, b.reshape(1, D))
